# baseline (speedup 1.0000x reference)
import functools

import jax
import jax.numpy as jnp
from jax.experimental import pallas as pl
from jax.experimental.pallas import tpu as pltpu

_B = 32
_HID = 32
_G = 32


def _silu(x):
    h = x * 0.5
    return h + h * jnp.tanh(h)


def _dot(a, b):
    return jax.lax.dot_general(a, b, (((1,), (0,)), ((), ())),
                               preferred_element_type=jnp.float32)


def _ln_mm(x, w, b, jmat):
    mu = _dot(x, jmat)
    xc = x - mu
    var = _dot(xc * xc, jmat)
    return xc * jax.lax.rsqrt(var + 1e-5) * w + b


def _build_upsampled(x, w, scratch):
    prev = jnp.concatenate([x[:, 0:1, 0:w], x[:, 0:15, 3 * w:4 * w]], axis=1)
    n0 = x[:, :, 0:w]
    n1 = x[:, :, w:2 * w]
    n2 = x[:, :, 2 * w:3 * w]
    n3 = x[:, :, 3 * w:4 * w]
    even = jnp.concatenate([(prev + n0) * 0.5, n0, (n0 + n1) * 0.5, n1], axis=2)
    odd = jnp.concatenate([(n1 + n2) * 0.5, n2, (n2 + n3) * 0.5, n3], axis=2)
    scratch[:, 0:32:2, :] = even
    scratch[:, 1:32:2, :] = odd
    last = x[:, 15:16, 3 * w:4 * w]
    scratch[:, 32:33, :] = jnp.concatenate([last, last, last, last], axis=2)
    return scratch[...]


def _cols(u, w, rows):
    c0 = u[:, 0:32, :]
    c1 = jnp.concatenate([u[:, 0:32, w:4 * w], u[:, 1:33, 0:w]], axis=2)
    c2 = jnp.concatenate([u[:, 0:32, 2 * w:4 * w], u[:, 1:33, 0:2 * w]], axis=2)
    return [r.reshape(rows, 4 * w) for r in (c0, c1, c2)]


def _rotmats():
    i = jax.lax.broadcasted_iota(jnp.int32, (3, 3), 0)
    j = jax.lax.broadcasted_iota(jnp.int32, (3, 3), 1)
    p1 = (i == (j + 1) % 3).astype(jnp.float32)
    p2 = (i == (j + 2) % 3).astype(jnp.float32)
    return _bd4(p1), _bd4(p2)


def _bd4(w):
    r, c = w.shape
    z = jnp.zeros((r, c), jnp.float32)
    rows = [
        jnp.concatenate([w, z, z, z], axis=1),
        jnp.concatenate([z, w, z, z], axis=1),
        jnp.concatenate([z, z, w, z], axis=1),
        jnp.concatenate([z, z, z, w], axis=1),
    ]
    return jnp.concatenate(rows, axis=0)


def _til4(row):
    return jnp.concatenate([row, row, row, row], axis=1)


def _egnn_pool_kernel(t_ref, c_ref,
                      em_w1r, em_b1r, em_w2r, em_b2r, em_w3r, em_b3r,
                      lne_wr, lne_br, ei_wr, ei_br,
                      ee_w1r, ee_b1r, ee_w2r, ee_b2r,
                      ec_w1r, ec_b1r, ec_w2r, ex_w1r, ex_b1r, ex_w2r,
                      en_w1r, en_b1r, en_w2r, en_b2r,
                      eo_wr, eo_br, lnh_wr, lnh_br,
                      h_out, x_out, hscr, cscr):
    C = _HID
    G = t_ref.shape[0]
    R = G * 32

    em_w1_full = em_w1r[...]
    ee_w1_full = ee_w1r[...]
    en_w1_full = en_w1r[...]
    em_w1a = _bd4(em_w1_full[0:C])
    em_w1b = _bd4(em_w1_full[C:2 * C])
    em_w2 = _bd4(em_w2r[...])
    em_w3 = _bd4(em_w3r[...])
    ei_w = _bd4(ei_wr[...])
    ee_w1c = ee_w1_full[C:2 * C]
    wc = _bd4(_dot(ei_wr[...], ee_w1c))
    ee_w1h = _bd4(ee_w1_full[0:C])
    ee_w1e = _bd4(ee_w1_full[2 * C + 1:3 * C + 1])
    ee_w2 = _bd4(ee_w2r[...])
    ec_w1 = _bd4(ec_w1r[...])
    ex_w1 = _bd4(ex_w1r[...])
    en_w1a = _bd4(en_w1_full[0:C])
    en_w1b = _bd4(en_w1_full[C:2 * C])
    en_w2 = _bd4(en_w2r[...])
    eo_w = _bd4(eo_wr[...])

    em_b1 = _til4(em_b1r[...]); em_b2 = _til4(em_b2r[...])
    em_b3 = _til4(em_b3r[...])
    lne_w = _til4(lne_wr[...]); lne_b = _til4(lne_br[...])
    ei_b = _til4(ei_br[...])
    bch = _til4(_dot(ei_br[...], ee_w1c) + ee_b1r[...])
    ee_b2 = _til4(ee_b2r[...])
    ec_b1 = _til4(ec_b1r[...]); ex_b1 = _til4(ex_b1r[...])
    en_b1 = _til4(en_b1r[...]); en_b2 = _til4(en_b2r[...])
    eo_b = _til4(eo_br[...])
    lnh_w = _til4(lnh_wr[...]); lnh_b = _til4(lnh_br[...])
    w1d = _til4(ee_w1_full[2 * C:2 * C + 1])

    jmat = _bd4(jnp.full((C, C), 1.0 / C, jnp.float32))
    dmat = _bd4(jnp.full((3, C), 1.0, jnp.float32))
    o33 = _bd4(jnp.full((3, 3), 1.0, jnp.float32))
    ecw2f = _bd4(jnp.concatenate([ec_w2r[...]] * 3, axis=1))
    exw2f = _bd4(jnp.concatenate([ex_w2r[...]] * 3, axis=1))

    huf = _build_upsampled(t_ref[...], _HID, hscr)
    cuf = _build_upsampled(c_ref[...], 3, cscr)

    hcols = _cols(huf, _HID, R)
    xcols = _cols(cuf, 3, R)

    hp = (hcols[0] + hcols[1] + hcols[2]) * (1.0 / 3.0)
    xp = (xcols[0] + xcols[1] + xcols[2]) * (1.0 / 3.0)
    P1, P2 = _rotmats()
    xp_r1 = _dot(xp, P1)
    xp_r2 = _dot(xp, P2)

    hr = _dot(hp, ei_w) + ei_b
    hp_em = _dot(hp, em_w1a)
    hr_ee = _dot(hr, ee_w1h)

    agg = jnp.zeros_like(hr)
    xacc = xp
    for k in range(3):
        hc_raw = hcols[k]
        xc = xcols[k]
        ea = jnp.maximum(hp_em + _dot(hc_raw, em_w1b) + em_b1, 0.0)
        ea = jnp.maximum(_dot(ea, em_w2) + em_b2, 0.0)
        ea = _dot(ea, em_w3) + em_b3
        ea = _ln_mm(ea, lne_w, lne_b, jmat)
        diff = xp - xc
        dsq = diff * diff
        d2c = _dot(dsq, dmat)
        d23 = _dot(dsq, o33)
        ds = jnp.sqrt(d23 + 1e-8) + 1.0
        dn = diff * jax.lax.rsqrt(ds * ds)
        m = _silu(hr_ee + _dot(hc_raw, wc) + d2c * w1d
                  + _dot(ea, ee_w1e) + bch)
        m = _silu(_dot(m, ee_w2) + ee_b2)
        tcoef = _dot(_silu(_dot(m, ec_w1) + ec_b1), ecw2f)
        xcoef = _dot(_silu(_dot(m, ex_w1) + ex_b1), exw2f)
        cr = xp_r1 * _dot(xc, P2) - xp_r2 * _dot(xc, P1)
        crn2 = _dot(cr * cr, o33)
        cs = jnp.sqrt(crn2) + 1.0
        cr = cr * jax.lax.rsqrt(cs * cs)
        xacc = xacc + dn * tcoef + cr * xcoef
        agg = agg + m

    h2 = hr + _dot(_silu(_dot(hr, en_w1a) + _dot(agg, en_w1b)
                         + en_b1), en_w2) + en_b2
    h2 = _dot(h2, eo_w) + eo_b
    h2 = _ln_mm(h2, lnh_w, lnh_b, jmat)

    h_out[...] = h2.reshape(G, 32, 4 * _HID)
    x_out[...] = xacc.reshape(G, 32, 12)


@jax.jit
def _run(h, coords, p):
    B, C = _B, _HID
    N = h.shape[0] // B
    t = h.reshape(B, N // 4, 4 * C)
    c = coords.reshape(B, N // 4, 12)

    def v(name):
        return p[name].reshape(1, -1)

    weights = [
        p['em_w1'], v('em_b1'), p['em_w2'], v('em_b2'), p['em_w3'], v('em_b3'),
        v('lne_w'), v('lne_b'), p['ei_w'], v('ei_b'),
        p['ee_w1'], v('ee_b1'), p['ee_w2'], v('ee_b2'),
        p['ec_w1'], v('ec_b1'), p['ec_w2'],
        p['ex_w1'], v('ex_b1'), p['ex_w2'],
        p['en_w1'], v('en_b1'), p['en_w2'], v('en_b2'),
        p['eo_w'], v('eo_b'), v('lnh_w'), v('lnh_b'),
    ]

    grid = (B // _G,)
    w_specs = [pl.BlockSpec(w.shape, lambda i: (0, 0)) for w in weights]
    h_out, x_out = pl.pallas_call(
        _egnn_pool_kernel,
        grid=grid,
        in_specs=[
            pl.BlockSpec((_G, N // 4, 4 * C), lambda i: (i, 0, 0)),
            pl.BlockSpec((_G, N // 4, 12), lambda i: (i, 0, 0)),
        ] + w_specs,
        out_specs=[
            pl.BlockSpec((_G, 32, 4 * C), lambda i: (i, 0, 0)),
            pl.BlockSpec((_G, 32, 12), lambda i: (i, 0, 0)),
        ],
        out_shape=[
            jax.ShapeDtypeStruct((B, 32, 4 * C), jnp.float32),
            jax.ShapeDtypeStruct((B, 32, 12), jnp.float32),
        ],
        scratch_shapes=[
            pltpu.VMEM((_G, 33, 4 * C), jnp.float32),
            pltpu.VMEM((_G, 33, 12), jnp.float32),
        ],
    )(t, c, *weights)
    return h_out.reshape(B * 2 * N, C), x_out.reshape(B * 2 * N, 3)


def kernel(h, coords, batch, params):
    del batch
    return _run(h, coords, params)


# speedup vs baseline: 1.0037x; 1.0037x over previous
import jax
import jax.numpy as jnp
from jax.experimental import pallas as pl
from jax.experimental.pallas import tpu as pltpu

_B = 32
_HID = 32
_G = 32


def _silu(x):
    h = x * 0.5
    return h + h * jnp.tanh(h)


def _dot(a, b):
    return jax.lax.dot_general(a, b, (((1,), (0,)), ((), ())),
                               preferred_element_type=jnp.float32)


def _ln_mm(x, w, b, jmat):
    mu = _dot(x, jmat)
    xc = x - mu
    var = _dot(xc * xc, jmat)
    return xc * jax.lax.rsqrt(var + 1e-5) * w + b


def _build_upsampled(x, w, scratch):
    prev = jnp.concatenate([x[:, 0:1, 0:w], x[:, 0:15, 3 * w:4 * w]], axis=1)
    n0 = x[:, :, 0:w]
    n1 = x[:, :, w:2 * w]
    n2 = x[:, :, 2 * w:3 * w]
    n3 = x[:, :, 3 * w:4 * w]
    even = jnp.concatenate([(prev + n0) * 0.5, n0, (n0 + n1) * 0.5, n1], axis=2)
    odd = jnp.concatenate([(n1 + n2) * 0.5, n2, (n2 + n3) * 0.5, n3], axis=2)
    scratch[:, 0:32:2, :] = even
    scratch[:, 1:32:2, :] = odd
    last = x[:, 15:16, 3 * w:4 * w]
    scratch[:, 32:33, :] = jnp.concatenate([last, last, last, last], axis=2)
    return scratch[...]


def _cols(u, w, rows):
    c0 = u[:, 0:32, :]
    c1 = jnp.concatenate([u[:, 0:32, w:4 * w], u[:, 1:33, 0:w]], axis=2)
    c2 = jnp.concatenate([u[:, 0:32, 2 * w:4 * w], u[:, 1:33, 0:2 * w]], axis=2)
    return [r.reshape(rows, 4 * w) for r in (c0, c1, c2)]


def _rotmats():
    i = jax.lax.broadcasted_iota(jnp.int32, (3, 3), 0)
    j = jax.lax.broadcasted_iota(jnp.int32, (3, 3), 1)
    p1 = (i == (j + 1) % 3).astype(jnp.float32)
    p2 = (i == (j + 2) % 3).astype(jnp.float32)
    return _bd4(p1), _bd4(p2)


def _bd4(w):
    r, c = w.shape
    z = jnp.zeros((r, c), jnp.float32)
    rows = [
        jnp.concatenate([w, z, z, z], axis=1),
        jnp.concatenate([z, w, z, z], axis=1),
        jnp.concatenate([z, z, w, z], axis=1),
        jnp.concatenate([z, z, z, w], axis=1),
    ]
    return jnp.concatenate(rows, axis=0)


def _til4(row):
    return jnp.concatenate([row, row, row, row], axis=1)


def _egnn_pool_kernel(t_ref, c_ref,
                      em_w1r, em_b1r, em_w2r, em_b2r, em_w3r, em_b3r,
                      lne_wr, lne_br, ei_wr, ei_br,
                      ee_w1r, ee_b1r, ee_w2r, ee_b2r,
                      ec_w1r, ec_b1r, ec_w2r, ex_w1r, ex_b1r, ex_w2r,
                      en_w1r, en_b1r, en_w2r, en_b2r,
                      eo_wr, eo_br, lnh_wr, lnh_br,
                      h_out, x_out, hscr, cscr):
    C = _HID
    G = t_ref.shape[0]
    R = G * 32

    em_w1_full = em_w1r[...]
    ee_w1_full = ee_w1r[...]
    en_w1_full = en_w1r[...]
    em_w1a = _bd4(em_w1_full[0:C])
    em_w1b = _bd4(em_w1_full[C:2 * C])
    em_w2 = _bd4(em_w2r[...])
    em_w3 = _bd4(em_w3r[...])
    ei_w = _bd4(ei_wr[...])
    ee_w1c = ee_w1_full[C:2 * C]
    wc = _bd4(_dot(ei_wr[...], ee_w1c))
    ee_w1h = _bd4(ee_w1_full[0:C])
    ee_w1e = _bd4(ee_w1_full[2 * C + 1:3 * C + 1])
    ee_w2 = _bd4(ee_w2r[...])
    ec_w1 = _bd4(ec_w1r[...])
    ex_w1 = _bd4(ex_w1r[...])
    en_w1a = _bd4(en_w1_full[0:C])
    en_w1b = _bd4(en_w1_full[C:2 * C])
    en_w2 = _bd4(en_w2r[...])
    eo_w = _bd4(eo_wr[...])

    em_b1 = _til4(em_b1r[...]); em_b2 = _til4(em_b2r[...])
    em_b3 = _til4(em_b3r[...])
    lne_w = _til4(lne_wr[...]); lne_b = _til4(lne_br[...])
    ei_b = _til4(ei_br[...])
    bch = _til4(_dot(ei_br[...], ee_w1c) + ee_b1r[...])
    ee_b2 = _til4(ee_b2r[...])
    ec_b1 = _til4(ec_b1r[...]); ex_b1 = _til4(ex_b1r[...])
    en_b1 = _til4(en_b1r[...]); en_b2 = _til4(en_b2r[...])
    eo_b = _til4(eo_br[...])
    lnh_w = _til4(lnh_wr[...]); lnh_b = _til4(lnh_br[...])
    w1d = _til4(ee_w1_full[2 * C:2 * C + 1])

    jmat = _bd4(jnp.full((C, C), 1.0 / C, jnp.float32))
    dmat = _bd4(jnp.full((3, C), 1.0, jnp.float32))
    o33 = _bd4(jnp.full((3, 3), 1.0, jnp.float32))
    ecw2f = _bd4(jnp.concatenate([ec_w2r[...]] * 3, axis=1))
    exw2f = _bd4(jnp.concatenate([ex_w2r[...]] * 3, axis=1))

    huf = _build_upsampled(t_ref[...], _HID, hscr)
    cuf = _build_upsampled(c_ref[...], 3, cscr)

    hcols = _cols(huf, _HID, R)
    xcols = _cols(cuf, 3, R)

    hp = (hcols[0] + hcols[1] + hcols[2]) * (1.0 / 3.0)
    xp = (xcols[0] + xcols[1] + xcols[2]) * (1.0 / 3.0)
    P1, P2 = _rotmats()
    xp_r1 = _dot(xp, P1)
    xp_r2 = _dot(xp, P2)

    hr = _dot(hp, ei_w) + ei_b
    hp_em = _dot(hp, em_w1a)
    hr_ee = _dot(hr, ee_w1h)

    agg = jnp.zeros_like(hr)
    xacc = xp
    for k in range(3):
        hc_raw = hcols[k]
        xc = xcols[k]
        ea = jnp.maximum(hp_em + _dot(hc_raw, em_w1b) + em_b1, 0.0)
        ea = jnp.maximum(_dot(ea, em_w2) + em_b2, 0.0)
        ea = _dot(ea, em_w3) + em_b3
        ea = _ln_mm(ea, lne_w, lne_b, jmat)
        diff = xp - xc
        dsq = diff * diff
        d2c = _dot(dsq, dmat)
        d23 = _dot(dsq, o33)
        ds = jnp.sqrt(d23 + 1e-8) + 1.0
        dn = diff * jax.lax.rsqrt(ds * ds)
        m = _silu(hr_ee + _dot(hc_raw, wc) + d2c * w1d
                  + _dot(ea, ee_w1e) + bch)
        m = _silu(_dot(m, ee_w2) + ee_b2)
        tcoef = _dot(_silu(_dot(m, ec_w1) + ec_b1), ecw2f)
        xcoef = _dot(_silu(_dot(m, ex_w1) + ex_b1), exw2f)
        cr = xp_r1 * _dot(xc, P2) - xp_r2 * _dot(xc, P1)
        crn2 = _dot(cr * cr, o33)
        cs = jnp.sqrt(crn2) + 1.0
        cr = cr * jax.lax.rsqrt(cs * cs)
        xacc = xacc + dn * tcoef + cr * xcoef
        agg = agg + m

    h2 = hr + _dot(_silu(_dot(hr, en_w1a) + _dot(agg, en_w1b)
                         + en_b1), en_w2) + en_b2
    h2 = _dot(h2, eo_w) + eo_b
    h2 = _ln_mm(h2, lnh_w, lnh_b, jmat)

    h_out[...] = h2.reshape(G, 32, 4 * _HID)
    x_out[...] = xacc.reshape(G, 32, 12)


@jax.jit
def _run(h, coords, p):
    B, C = _B, _HID
    N = h.shape[0] // B
    t = h.reshape(B, N // 4, 4 * C)
    c = coords.reshape(B, N // 4, 12)

    def v(name):
        return p[name].reshape(1, -1)

    weights = [
        p['em_w1'], v('em_b1'), p['em_w2'], v('em_b2'), p['em_w3'], v('em_b3'),
        v('lne_w'), v('lne_b'), p['ei_w'], v('ei_b'),
        p['ee_w1'], v('ee_b1'), p['ee_w2'], v('ee_b2'),
        p['ec_w1'], v('ec_b1'), p['ec_w2'],
        p['ex_w1'], v('ex_b1'), p['ex_w2'],
        p['en_w1'], v('en_b1'), p['en_w2'], v('en_b2'),
        p['eo_w'], v('eo_b'), v('lnh_w'), v('lnh_b'),
    ]

    grid = (B // _G,)
    w_specs = [pl.BlockSpec(w.shape, lambda i: (0, 0)) for w in weights]
    h_out, x_out = pl.pallas_call(
        _egnn_pool_kernel,
        grid=grid,
        in_specs=[
            pl.BlockSpec((_G, N // 4, 4 * C), lambda i: (i, 0, 0)),
            pl.BlockSpec((_G, N // 4, 12), lambda i: (i, 0, 0)),
        ] + w_specs,
        out_specs=[
            pl.BlockSpec((_G, 32, 4 * C), lambda i: (i, 0, 0)),
            pl.BlockSpec((_G, 32, 12), lambda i: (i, 0, 0)),
        ],
        out_shape=[
            jax.ShapeDtypeStruct((B, 32, 4 * C), jnp.float32),
            jax.ShapeDtypeStruct((B, 32, 12), jnp.float32),
        ],
        scratch_shapes=[
            pltpu.VMEM((_G, 33, 4 * C), jnp.float32),
            pltpu.VMEM((_G, 33, 12), jnp.float32),
        ],
    )(t, c, *weights)
    return h_out.reshape(B * 2 * N, C), x_out.reshape(B * 2 * N, 3)


def kernel(h, coords, batch, params):
    del batch
    return _run(h, coords, params)


# speedup vs baseline: 1.0113x; 1.0076x over previous
import jax
import jax.numpy as jnp
from jax.experimental import pallas as pl
from jax.experimental.pallas import tpu as pltpu

_B = 32
_HID = 32
_G = 32


def _silu(x):
    h = x * 0.5
    return h + h * jnp.tanh(h)


def _dot(a, b):
    return jax.lax.dot_general(a, b, (((1,), (0,)), ((), ())),
                               preferred_element_type=jnp.float32)


def _ln_mm(x, w, b, jmat):
    mu = _dot(x, jmat)
    xc = x - mu
    var = _dot(xc * xc, jmat)
    return xc * jax.lax.rsqrt(var + 1e-5) * w + b


def _build_upsampled(x, w, scratch):
    prev = jnp.concatenate([x[:, 0:1, 0:w], x[:, 0:15, 3 * w:4 * w]], axis=1)
    n0 = x[:, :, 0:w]
    n1 = x[:, :, w:2 * w]
    n2 = x[:, :, 2 * w:3 * w]
    n3 = x[:, :, 3 * w:4 * w]
    even = jnp.concatenate([(prev + n0) * 0.5, n0, (n0 + n1) * 0.5, n1], axis=2)
    odd = jnp.concatenate([(n1 + n2) * 0.5, n2, (n2 + n3) * 0.5, n3], axis=2)
    scratch[:, 0:32:2, :] = even
    scratch[:, 1:32:2, :] = odd
    last = x[:, 15:16, 3 * w:4 * w]
    scratch[:, 32:33, :] = jnp.concatenate([last, last, last, last], axis=2)
    return scratch[...]


def _cols(u, w, rows):
    c0 = u[:, 0:32, :]
    c1 = jnp.concatenate([u[:, 0:32, w:4 * w], u[:, 1:33, 0:w]], axis=2)
    c2 = jnp.concatenate([u[:, 0:32, 2 * w:4 * w], u[:, 1:33, 0:2 * w]], axis=2)
    return [r.reshape(rows, 4 * w) for r in (c0, c1, c2)]


def _rotmats():
    i = jax.lax.broadcasted_iota(jnp.int32, (3, 3), 0)
    j = jax.lax.broadcasted_iota(jnp.int32, (3, 3), 1)
    p1 = (i == (j + 1) % 3).astype(jnp.float32)
    p2 = (i == (j + 2) % 3).astype(jnp.float32)
    return _bd4(p1), _bd4(p2)


def _bd4(w):
    r, c = w.shape
    z = jnp.zeros((r, c), jnp.float32)
    rows = [
        jnp.concatenate([w, z, z, z], axis=1),
        jnp.concatenate([z, w, z, z], axis=1),
        jnp.concatenate([z, z, w, z], axis=1),
        jnp.concatenate([z, z, z, w], axis=1),
    ]
    return jnp.concatenate(rows, axis=0)


def _til4(row):
    return jnp.concatenate([row, row, row, row], axis=1)


def _bd3(w):
    r, c = w.shape
    z = jnp.zeros((r, c), jnp.float32)
    rows = [
        jnp.concatenate([w, z, z], axis=1),
        jnp.concatenate([z, w, z], axis=1),
        jnp.concatenate([z, z, w], axis=1),
    ]
    return jnp.concatenate(rows, axis=0)


def _egnn_pool_kernel(t_ref, c_ref,
                      em_w1r, em_b1r, em_w2r, em_b2r, em_w3r, em_b3r,
                      lne_wr, lne_br, ei_wr, ei_br,
                      ee_w1r, ee_b1r, ee_w2r, ee_b2r,
                      ec_w1r, ec_b1r, ec_w2r, ex_w1r, ex_b1r, ex_w2r,
                      en_w1r, en_b1r, en_w2r, en_b2r,
                      eo_wr, eo_br, lnh_wr, lnh_br,
                      h_out, x_out, hscr, cscr):
    C = _HID
    G = t_ref.shape[0]
    R = G * 32

    em_w1_full = em_w1r[...]
    ee_w1_full = ee_w1r[...]
    en_w1_full = en_w1r[...]
    em_w1a = _bd4(em_w1_full[0:C])
    em_w1b = _bd4(em_w1_full[C:2 * C])
    em_w2 = _bd4(em_w2r[...])
    em_w3 = _bd4(em_w3r[...])
    ei_w = _bd4(ei_wr[...])
    ee_w1c = ee_w1_full[C:2 * C]
    wc = _bd4(_dot(ei_wr[...], ee_w1c))
    ee_w1h = _bd4(ee_w1_full[0:C])
    ee_w1e = _bd4(ee_w1_full[2 * C + 1:3 * C + 1])
    ee_w2 = _bd4(ee_w2r[...])
    ec_w1 = _bd4(ec_w1r[...])
    ex_w1 = _bd4(ex_w1r[...])
    en_w1a = _bd4(en_w1_full[0:C])
    en_w1b = _bd4(en_w1_full[C:2 * C])
    en_w2 = _bd4(en_w2r[...])
    eo_w = _bd4(eo_wr[...])

    em_b1 = _til4(em_b1r[...]); em_b2 = _til4(em_b2r[...])
    em_b3 = _til4(em_b3r[...])
    lne_w = _til4(lne_wr[...]); lne_b = _til4(lne_br[...])
    ei_b = _til4(ei_br[...])
    bch = _til4(_dot(ei_br[...], ee_w1c) + ee_b1r[...])
    ee_b2 = _til4(ee_b2r[...])
    ec_b1 = _til4(ec_b1r[...]); ex_b1 = _til4(ex_b1r[...])
    en_b1 = _til4(en_b1r[...]); en_b2 = _til4(en_b2r[...])
    eo_b = _til4(eo_br[...])
    lnh_w = _til4(lnh_wr[...]); lnh_b = _til4(lnh_br[...])
    w1d = _til4(ee_w1_full[2 * C:2 * C + 1])

    jmat = _bd4(jnp.full((C, C), 1.0 / C, jnp.float32))
    dmat = _bd4(jnp.full((3, C), 1.0, jnp.float32))
    o33 = _bd4(jnp.full((3, 3), 1.0, jnp.float32))
    ecw2f = _bd4(jnp.concatenate([ec_w2r[...]] * 3, axis=1))
    exw2f = _bd4(jnp.concatenate([ex_w2r[...]] * 3, axis=1))

    huf = _build_upsampled(t_ref[...], _HID, hscr)
    cuf = _build_upsampled(c_ref[...], 3, cscr)

    hcols = _cols(huf, _HID, R)
    xcols = _cols(cuf, 3, R)

    hp = (hcols[0] + hcols[1] + hcols[2]) * (1.0 / 3.0)
    xp = (xcols[0] + xcols[1] + xcols[2]) * (1.0 / 3.0)

    P1, P2 = _rotmats()
    o33_9 = _bd3(o33)
    P1_9, P2_9 = _bd3(P1), _bd3(P2)
    z12 = jnp.zeros((12, 4 * C), jnp.float32)
    dmats = [jnp.concatenate([dmat if i == k else z12 for i in range(3)],
                             axis=0) for k in range(3)]
    xc_cat = jnp.concatenate(xcols, axis=1)
    xp3 = jnp.concatenate([xp, xp, xp], axis=1)
    diff_cat = xp3 - xc_cat
    dsq_cat = diff_cat * diff_cat
    d23 = _dot(dsq_cat, o33_9)
    dst = jnp.sqrt(d23 + 1e-8) + 1.0
    dn_cat = diff_cat * jax.lax.rsqrt(dst * dst)
    cr_cat = (_dot(xp3, P1_9) * _dot(xc_cat, P2_9)
              - _dot(xp3, P2_9) * _dot(xc_cat, P1_9))
    crn2 = _dot(cr_cat * cr_cat, o33_9)
    cst = jnp.sqrt(crn2) + 1.0
    cr_cat = cr_cat * jax.lax.rsqrt(cst * cst)

    hr = _dot(hp, ei_w) + ei_b
    hp_em = _dot(hp, em_w1a)
    hr_ee = _dot(hr, ee_w1h)

    agg = jnp.zeros_like(hr)
    tcoefs, xcoefs = [], []
    for k in range(3):
        hc_raw = hcols[k]
        ea = jnp.maximum(hp_em + _dot(hc_raw, em_w1b) + em_b1, 0.0)
        ea = jnp.maximum(_dot(ea, em_w2) + em_b2, 0.0)
        ea = _dot(ea, em_w3) + em_b3
        ea = _ln_mm(ea, lne_w, lne_b, jmat)
        d2c = _dot(dsq_cat, dmats[k])
        m = _silu(hr_ee + _dot(hc_raw, wc) + d2c * w1d
                  + _dot(ea, ee_w1e) + bch)
        m = _silu(_dot(m, ee_w2) + ee_b2)
        tcoefs.append(_dot(_silu(_dot(m, ec_w1) + ec_b1), ecw2f))
        xcoefs.append(_dot(_silu(_dot(m, ex_w1) + ex_b1), exw2f))
        agg = agg + m

    i12 = jax.lax.broadcasted_iota(jnp.int32, (12, 12), 0)
    j12 = jax.lax.broadcasted_iota(jnp.int32, (12, 12), 1)
    eye12 = (i12 == j12).astype(jnp.float32)
    ksum = jnp.concatenate([eye12, eye12, eye12], axis=0)
    contrib = (dn_cat * jnp.concatenate(tcoefs, axis=1)
               + cr_cat * jnp.concatenate(xcoefs, axis=1))
    xacc = xp + _dot(contrib, ksum)

    h2 = hr + _dot(_silu(_dot(hr, en_w1a) + _dot(agg, en_w1b)
                         + en_b1), en_w2) + en_b2
    h2 = _dot(h2, eo_w) + eo_b
    h2 = _ln_mm(h2, lnh_w, lnh_b, jmat)

    h_out[...] = h2.reshape(G, 32, 4 * _HID)
    x_out[...] = xacc.reshape(G, 32, 12)


@jax.jit
def _run(h, coords, p):
    B, C = _B, _HID
    N = h.shape[0] // B
    t = h.reshape(B, N // 4, 4 * C)
    c = coords.reshape(B, N // 4, 12)

    def v(name):
        return p[name].reshape(1, -1)

    weights = [
        p['em_w1'], v('em_b1'), p['em_w2'], v('em_b2'), p['em_w3'], v('em_b3'),
        v('lne_w'), v('lne_b'), p['ei_w'], v('ei_b'),
        p['ee_w1'], v('ee_b1'), p['ee_w2'], v('ee_b2'),
        p['ec_w1'], v('ec_b1'), p['ec_w2'],
        p['ex_w1'], v('ex_b1'), p['ex_w2'],
        p['en_w1'], v('en_b1'), p['en_w2'], v('en_b2'),
        p['eo_w'], v('eo_b'), v('lnh_w'), v('lnh_b'),
    ]

    grid = (B // _G,)
    w_specs = [pl.BlockSpec(w.shape, lambda i: (0, 0)) for w in weights]
    h_out, x_out = pl.pallas_call(
        _egnn_pool_kernel,
        grid=grid,
        in_specs=[
            pl.BlockSpec((_G, N // 4, 4 * C), lambda i: (i, 0, 0)),
            pl.BlockSpec((_G, N // 4, 12), lambda i: (i, 0, 0)),
        ] + w_specs,
        out_specs=[
            pl.BlockSpec((_G, 32, 4 * C), lambda i: (i, 0, 0)),
            pl.BlockSpec((_G, 32, 12), lambda i: (i, 0, 0)),
        ],
        out_shape=[
            jax.ShapeDtypeStruct((B, 32, 4 * C), jnp.float32),
            jax.ShapeDtypeStruct((B, 32, 12), jnp.float32),
        ],
        scratch_shapes=[
            pltpu.VMEM((_G, 33, 4 * C), jnp.float32),
            pltpu.VMEM((_G, 33, 12), jnp.float32),
        ],
    )(t, c, *weights)
    return h_out.reshape(B * 2 * N, C), x_out.reshape(B * 2 * N, 3)


def kernel(h, coords, batch, params):
    del batch
    return _run(h, coords, params)
